# baseline (speedup 1.0000x reference)
import functools

import jax
import jax.numpy as jnp
from jax import lax
from jax.experimental import pallas as pl
from jax.experimental.pallas import tpu as pltpu
from jax.experimental.pallas import tpu_sc as plsc

N = 10000
NP = N + 16
D = 128
NT = 3
DH = 48
DC = D - 2 * DH
E = 320000
NC, NS = 2, 16
NW = NC * NS
EPT = E // NW
CHR = 125
CH = 128
NCH = EPT // CHR
RPT = 624
RB = 10
BR = N // RB

NB = 5
LOOK = 3
UNR = 8


def _prep_body(x_ref, wsrc_ref, wdst_ref, aw_ref, ta_ref, tb_ref, tc_ref,
               ssrc_ref, sdst_ref):
    x = x_ref[...]
    h = jnp.dot(x, wsrc_ref[...], preferred_element_type=jnp.float32)
    a1 = aw_ref[0:D, :]
    a2 = aw_ref[D:2 * D, :]
    ssrc_ref[...] = jnp.dot(h, a1, preferred_element_type=jnp.float32)
    w2 = jnp.dot(wdst_ref[...], a2, preferred_element_type=jnp.float32)
    sdst_ref[...] = jnp.dot(x, w2, preferred_element_type=jnp.float32)
    ta_ref[...] = h[:, 0:DH]
    tb_ref[...] = h[:, DH:2 * DH]
    col = lax.broadcasted_iota(jnp.int32, (x.shape[0], DH - DC), 1)
    pad = jnp.where(col == 0, 1.0, 0.0).astype(jnp.float32)
    tc_ref[...] = jnp.concatenate([h[:, 2 * DH:D], pad], axis=1)


_prep = pl.pallas_call(
    _prep_body,
    grid=(RB,),
    in_specs=[
        pl.BlockSpec((BR, D), lambda i: (i, 0)),
        pl.BlockSpec((D, D), lambda i: (0, 0)),
        pl.BlockSpec((D, D), lambda i: (0, 0)),
        pl.BlockSpec((3 * D, 1), lambda i: (0, 0)),
    ],
    out_specs=[
        pl.BlockSpec((BR, DH), lambda i: (i, 0)),
        pl.BlockSpec((BR, DH), lambda i: (i, 0)),
        pl.BlockSpec((BR, DH), lambda i: (i, 0)),
        pl.BlockSpec((BR, 1), lambda i: (i, 0)),
        pl.BlockSpec((BR, 1), lambda i: (i, 0)),
    ],
    out_shape=[
        jax.ShapeDtypeStruct((N, DH), jnp.float32),
        jax.ShapeDtypeStruct((N, DH), jnp.float32),
        jax.ShapeDtypeStruct((N, DH), jnp.float32),
        jax.ShapeDtypeStruct((N, 1), jnp.float32),
        jax.ShapeDtypeStruct((N, 1), jnp.float32),
    ],
)


def _edge_body(ta_hbm, tb_hbm, tc_hbm, ssrc_hbm, sdst_hbm, src_hbm, dst_hbm,
               out_hbm, ssrc_v, sdst_v, src_v, dst_v, att_v, rows_v, zero_v,
               acc_sh, sem_g, sem_s, sem_z):
    ci = lax.axis_index("c")
    si = lax.axis_index("s")

    pltpu.async_copy(ssrc_hbm, ssrc_v.at[pl.ds(0, N)], sem_z)
    pltpu.async_copy(sdst_hbm, sdst_v.at[pl.ds(0, N)], sem_z)
    pltpu.async_copy(src_hbm.at[ci, si], src_v, sem_z)
    pltpu.async_copy(dst_hbm.at[ci, si], dst_v, sem_z)

    zrow = jnp.zeros((16,), jnp.float32)
    for i in range(48):
        for q in range(DH // 16):
            zero_v[i, pl.ds(q * 16, 16)] = zrow

    pltpu.make_async_copy(ssrc_hbm, ssrc_v.at[pl.ds(0, N)], sem_z).wait()
    pltpu.make_async_copy(sdst_hbm, sdst_v.at[pl.ds(0, N)], sem_z).wait()
    pltpu.make_async_copy(src_hbm.at[ci, si], src_v, sem_z).wait()
    pltpu.make_async_copy(dst_hbm.at[ci, si], dst_v, sem_z).wait()

    base = si * RPT

    for slab, tab_hbm in ((0, ta_hbm), (1, tb_hbm), (2, tc_hbm)):
        for i in range(RPT // 48):
            pltpu.async_copy(zero_v, acc_sh.at[pl.ds(base + i * 48, 48)], sem_z)

        @pl.when(si == NS - 1)
        def _ztail():
            pltpu.async_copy(zero_v.at[pl.ds(0, NP - RPT * NS)],
                             acc_sh.at[pl.ds(RPT * NS, NP - RPT * NS)], sem_z)

        for i in range(RPT // 48):
            pltpu.make_async_copy(zero_v,
                                  acc_sh.at[pl.ds(base + i * 48, 48)],
                                  sem_z).wait()

        @pl.when(si == NS - 1)
        def _ztailw():
            pltpu.make_async_copy(zero_v.at[pl.ds(0, NP - RPT * NS)],
                                  acc_sh.at[pl.ds(RPT * NS, NP - RPT * NS)],
                                  sem_z).wait()

        plsc.subcore_barrier()

        for b in range(LOOK):
            pltpu.async_copy(tab_hbm.at[src_v.at[b]], rows_v.at[b],
                             sem_g.at[b])

        def _group(jo, carry):
            for b in range(NB):
                j = jo * NB + b
                if slab == 0:
                    for k in range(CH // 16):
                        s16 = src_v[j, pl.ds(k * 16, 16)]
                        d16 = dst_v[j, pl.ds(k * 16, 16)]
                        x = (plsc.load_gather(ssrc_v, [s16])
                             + plsc.load_gather(sdst_v, [d16]))
                        x = jnp.where(x >= 0, x, 0.2 * x)
                        att_v[j, pl.ds(k * 16, 16)] = jnp.exp(x)
                pltpu.make_async_copy(tab_hbm.at[src_v.at[j]], rows_v.at[b],
                                      sem_g.at[b]).wait()
                jj = jnp.full((16,), j, jnp.int32)

                def _mul(i, carry2):
                    e0 = i * UNR
                    for u in range(UNR):
                        ee = jnp.full((16,), e0 + u, jnp.int32)
                        a16 = plsc.load_gather(att_v, [jj, ee])
                        for q in range(DH // 16):
                            rows_v[b, e0 + u, pl.ds(q * 16, 16)] = (
                                rows_v[b, e0 + u, pl.ds(q * 16, 16)] * a16)
                    return carry2

                lax.fori_loop(0, CH // UNR, _mul, 0)
                pltpu.async_copy(rows_v.at[b], acc_sh.at[dst_v.at[j]],
                                 sem_s.at[b], add=True)
                bb = (b + LOOK) % NB

                @pl.when(j >= NB - LOOK)
                def _drain():
                    pltpu.make_async_copy(rows_v.at[bb],
                                          acc_sh.at[dst_v.at[0]],
                                          sem_s.at[bb]).wait()

                @pl.when(j + LOOK <= NCH - 1)
                def _prefetch():
                    pltpu.async_copy(tab_hbm.at[src_v.at[j + LOOK]],
                                     rows_v.at[bb], sem_g.at[bb])
            return carry

        lax.fori_loop(0, NCH // NB, _group, 0)

        for c in range(NCH - (NB - LOOK), NCH):
            pltpu.make_async_copy(rows_v.at[c % NB], acc_sh.at[dst_v.at[0]],
                                  sem_s.at[c % NB]).wait()

        plsc.subcore_barrier()

        pltpu.sync_copy(acc_sh.at[pl.ds(base, RPT)],
                        out_hbm.at[slab, ci, pl.ds(base, RPT)])

        @pl.when(si == NS - 1)
        def _tail():
            pltpu.sync_copy(acc_sh.at[pl.ds(RPT * NS, NP - RPT * NS)],
                            out_hbm.at[slab, ci, pl.ds(RPT * NS, NP - RPT * NS)])

        plsc.subcore_barrier()


@functools.cache
def _edge():
    return pl.kernel(
        _edge_body,
        out_type=jax.ShapeDtypeStruct((NT, NC, NP, DH), jnp.float32),
        mesh=plsc.VectorSubcoreMesh(core_axis_name="c", subcore_axis_name="s",
                                    num_cores=NC, num_subcores=NS),
        compiler_params=pltpu.CompilerParams(needs_layout_passes=False,
                                             use_tc_tiling_on_sc=False),
        scratch_types=[
            pltpu.VMEM((NP,), jnp.float32),
            pltpu.VMEM((NP,), jnp.float32),
            pltpu.VMEM((NCH, CH), jnp.int32),
            pltpu.VMEM((NCH, CH), jnp.int32),
            pltpu.VMEM((NCH, CH), jnp.float32),
            pltpu.VMEM((NB, CH, DH), jnp.float32),
            pltpu.VMEM((48, DH), jnp.float32),
            pltpu.VMEM_SHARED((NP, DH), jnp.float32),
            pltpu.SemaphoreType.DMA((NB,)),
            pltpu.SemaphoreType.DMA((NB,)),
            pltpu.SemaphoreType.DMA,
        ],
    )


def _post_body(p_ref, out_ref):
    a = p_ref[0, 0] + p_ref[0, 1]
    b = p_ref[1, 0] + p_ref[1, 1]
    c = p_ref[2, 0] + p_ref[2, 1]
    h = jnp.concatenate([a, b, c[:, 0:DC]], axis=1)
    sw = c[:, DC:DC + 1]
    r = h / (sw + 1e-8)
    out_ref[...] = jnp.where(r > 0, r, jnp.exp(r) - 1.0)


_post = pl.pallas_call(
    _post_body,
    grid=(RB,),
    in_specs=[pl.BlockSpec((NT, NC, BR, DH), lambda i: (0, 0, i, 0))],
    out_specs=pl.BlockSpec((BR, D), lambda i: (i, 0)),
    out_shape=jax.ShapeDtypeStruct((N, D), jnp.float32),
)


def kernel(source_vecs, edge_index, W_src, W_dst, W_rating, a_w):
    del W_rating
    src = edge_index[0].astype(jnp.int32).reshape(NC, NS, NCH, CHR)
    dst = edge_index[1].astype(jnp.int32).reshape(NC, NS, NCH, CHR)
    padw = ((0, 0), (0, 0), (0, 0), (0, CH - CHR))
    src = jnp.pad(src, padw)
    dump = jnp.broadcast_to((N + jnp.arange(NS, dtype=jnp.int32))
                            [None, :, None, None], (NC, NS, NCH, CH - CHR))
    dst = jnp.concatenate([dst, dump], axis=3)
    ta, tb, tc, ssrc, sdst = _prep(source_vecs, W_src, W_dst, a_w)
    partial = _edge()(ta, tb, tc, ssrc.reshape(N), sdst.reshape(N), src, dst)
    return _post(partial)


# speedup vs baseline: 1.6352x; 1.6352x over previous
import functools

import jax
import jax.numpy as jnp
from jax import lax
from jax.experimental import pallas as pl
from jax.experimental.pallas import tpu as pltpu
from jax.experimental.pallas import tpu_sc as plsc

N = 10000
NP = N
D = 128
NT = 3
DH = 48
DC = D - 2 * DH
E = 320000
NC, NS = 2, 16
NW = NC * NS
EPT = E // NW
CH = 80
NCH = EPT // CH
RPT = 624
RB = 10
BR = N // RB

NB = 5
LOOK = 3
UNR = 8


def _prep_body(x_ref, wsrc_ref, wdst_ref, aw_ref, ta_ref, tb_ref, tc_ref,
               ssrc_ref, sdst_ref):
    x = x_ref[...]
    h = jnp.dot(x, wsrc_ref[...], preferred_element_type=jnp.float32)
    a1 = aw_ref[0:D, :]
    a2 = aw_ref[D:2 * D, :]
    ssrc_ref[...] = jnp.dot(h, a1, preferred_element_type=jnp.float32)
    w2 = jnp.dot(wdst_ref[...], a2, preferred_element_type=jnp.float32)
    sdst_ref[...] = jnp.dot(x, w2, preferred_element_type=jnp.float32)
    ta_ref[...] = h[:, 0:DH]
    tb_ref[...] = h[:, DH:2 * DH]
    col = lax.broadcasted_iota(jnp.int32, (x.shape[0], DH - DC), 1)
    pad = jnp.where(col == 0, 1.0, 0.0).astype(jnp.float32)
    tc_ref[...] = jnp.concatenate([h[:, 2 * DH:D], pad], axis=1)


_prep = pl.pallas_call(
    _prep_body,
    grid=(RB,),
    in_specs=[
        pl.BlockSpec((BR, D), lambda i: (i, 0)),
        pl.BlockSpec((D, D), lambda i: (0, 0)),
        pl.BlockSpec((D, D), lambda i: (0, 0)),
        pl.BlockSpec((3 * D, 1), lambda i: (0, 0)),
    ],
    out_specs=[
        pl.BlockSpec((BR, DH), lambda i: (i, 0)),
        pl.BlockSpec((BR, DH), lambda i: (i, 0)),
        pl.BlockSpec((BR, DH), lambda i: (i, 0)),
        pl.BlockSpec((BR, 1), lambda i: (i, 0)),
        pl.BlockSpec((BR, 1), lambda i: (i, 0)),
    ],
    out_shape=[
        jax.ShapeDtypeStruct((N, DH), jnp.float32),
        jax.ShapeDtypeStruct((N, DH), jnp.float32),
        jax.ShapeDtypeStruct((N, DH), jnp.float32),
        jax.ShapeDtypeStruct((N, 1), jnp.float32),
        jax.ShapeDtypeStruct((N, 1), jnp.float32),
    ],
)


def _edge_body(ta_hbm, tb_hbm, tc_hbm, ssrc_hbm, sdst_hbm, src_hbm, dst_hbm,
               out_hbm, ssrc_v, sdst_v, src_v, dst_v, att_v, rows_v, zero_v,
               acc_sh, sem_g, sem_s, sem_z):
    ci = lax.axis_index("c")
    si = lax.axis_index("s")

    pltpu.async_copy(ssrc_hbm, ssrc_v.at[pl.ds(0, N)], sem_z)
    pltpu.async_copy(sdst_hbm, sdst_v.at[pl.ds(0, N)], sem_z)
    pltpu.async_copy(src_hbm.at[ci, si], src_v, sem_z)
    pltpu.async_copy(dst_hbm.at[ci, si], dst_v, sem_z)

    zrow = jnp.zeros((16,), jnp.float32)
    for i in range(48):
        for q in range(DH // 16):
            zero_v[i, pl.ds(q * 16, 16)] = zrow

    pltpu.make_async_copy(ssrc_hbm, ssrc_v.at[pl.ds(0, N)], sem_z).wait()
    pltpu.make_async_copy(sdst_hbm, sdst_v.at[pl.ds(0, N)], sem_z).wait()
    pltpu.make_async_copy(src_hbm.at[ci, si], src_v, sem_z).wait()
    pltpu.make_async_copy(dst_hbm.at[ci, si], dst_v, sem_z).wait()

    base = si * RPT

    for slab, tab_hbm in ((0, ta_hbm), (1, tb_hbm), (2, tc_hbm)):
        for i in range(RPT // 48):
            pltpu.async_copy(zero_v, acc_sh.at[pl.ds(base + i * 48, 48)], sem_z)

        @pl.when(si == NS - 1)
        def _ztail():
            pltpu.async_copy(zero_v.at[pl.ds(0, NP - RPT * NS)],
                             acc_sh.at[pl.ds(RPT * NS, NP - RPT * NS)], sem_z)

        for i in range(RPT // 48):
            pltpu.make_async_copy(zero_v,
                                  acc_sh.at[pl.ds(base + i * 48, 48)],
                                  sem_z).wait()

        @pl.when(si == NS - 1)
        def _ztailw():
            pltpu.make_async_copy(zero_v.at[pl.ds(0, NP - RPT * NS)],
                                  acc_sh.at[pl.ds(RPT * NS, NP - RPT * NS)],
                                  sem_z).wait()

        plsc.subcore_barrier()

        for b in range(LOOK):
            pltpu.async_copy(tab_hbm.at[src_v.at[b]], rows_v.at[b],
                             sem_g.at[b])

        def _group(jo, carry):
            for b in range(NB):
                j = jo * NB + b
                if slab == 0:
                    for k in range(CH // 16):
                        s16 = src_v[j, pl.ds(k * 16, 16)]
                        d16 = dst_v[j, pl.ds(k * 16, 16)]
                        x = (plsc.load_gather(ssrc_v, [s16])
                             + plsc.load_gather(sdst_v, [d16]))
                        x = jnp.where(x >= 0, x, 0.2 * x)
                        att_v[j, pl.ds(k * 16, 16)] = jnp.exp(x)
                pltpu.make_async_copy(tab_hbm.at[src_v.at[j]], rows_v.at[b],
                                      sem_g.at[b]).wait()
                jj = jnp.full((16,), j, jnp.int32)

                def _mul(i, carry2):
                    e0 = i * UNR
                    for u in range(UNR):
                        ee = jnp.full((16,), e0 + u, jnp.int32)
                        a16 = plsc.load_gather(att_v, [jj, ee])
                        for q in range(DH // 16):
                            rows_v[b, e0 + u, pl.ds(q * 16, 16)] = (
                                rows_v[b, e0 + u, pl.ds(q * 16, 16)] * a16)
                    return carry2

                lax.fori_loop(0, CH // UNR, _mul, 0)
                pltpu.async_copy(rows_v.at[b], acc_sh.at[dst_v.at[j]],
                                 sem_s.at[b], add=True)
                bb = (b + LOOK) % NB

                @pl.when(j >= NB - LOOK)
                def _drain():
                    pltpu.make_async_copy(rows_v.at[bb],
                                          acc_sh.at[dst_v.at[0]],
                                          sem_s.at[bb]).wait()

                @pl.when(j + LOOK <= NCH - 1)
                def _prefetch():
                    pltpu.async_copy(tab_hbm.at[src_v.at[j + LOOK]],
                                     rows_v.at[bb], sem_g.at[bb])
            return carry

        lax.fori_loop(0, NCH // NB, _group, 0)

        for c in range(NCH - (NB - LOOK), NCH):
            pltpu.make_async_copy(rows_v.at[c % NB], acc_sh.at[dst_v.at[0]],
                                  sem_s.at[c % NB]).wait()

        plsc.subcore_barrier()

        pltpu.sync_copy(acc_sh.at[pl.ds(base, RPT)],
                        out_hbm.at[slab, ci, pl.ds(base, RPT)])

        @pl.when(si == NS - 1)
        def _tail():
            pltpu.sync_copy(acc_sh.at[pl.ds(RPT * NS, NP - RPT * NS)],
                            out_hbm.at[slab, ci, pl.ds(RPT * NS, NP - RPT * NS)])

        plsc.subcore_barrier()


@functools.cache
def _edge():
    return pl.kernel(
        _edge_body,
        out_type=jax.ShapeDtypeStruct((NT, NC, NP, DH), jnp.float32),
        mesh=plsc.VectorSubcoreMesh(core_axis_name="c", subcore_axis_name="s",
                                    num_cores=NC, num_subcores=NS),
        compiler_params=pltpu.CompilerParams(needs_layout_passes=False,
                                             use_tc_tiling_on_sc=False),
        scratch_types=[
            pltpu.VMEM((NP,), jnp.float32),
            pltpu.VMEM((NP,), jnp.float32),
            pltpu.VMEM((NCH, CH), jnp.int32),
            pltpu.VMEM((NCH, CH), jnp.int32),
            pltpu.VMEM((NCH, CH), jnp.float32),
            pltpu.VMEM((NB, CH, DH), jnp.float32),
            pltpu.VMEM((48, DH), jnp.float32),
            pltpu.VMEM_SHARED((NP, DH), jnp.float32),
            pltpu.SemaphoreType.DMA((NB,)),
            pltpu.SemaphoreType.DMA((NB,)),
            pltpu.SemaphoreType.DMA,
        ],
    )


def _post_body(p_ref, out_ref):
    a = p_ref[0, 0] + p_ref[0, 1]
    b = p_ref[1, 0] + p_ref[1, 1]
    c = p_ref[2, 0] + p_ref[2, 1]
    h = jnp.concatenate([a, b, c[:, 0:DC]], axis=1)
    sw = c[:, DC:DC + 1]
    r = h / (sw + 1e-8)
    out_ref[...] = jnp.where(r > 0, r, jnp.exp(r) - 1.0)


_post = pl.pallas_call(
    _post_body,
    grid=(RB,),
    in_specs=[pl.BlockSpec((NT, NC, BR, DH), lambda i: (0, 0, i, 0))],
    out_specs=pl.BlockSpec((BR, D), lambda i: (i, 0)),
    out_shape=jax.ShapeDtypeStruct((N, D), jnp.float32),
)


def kernel(source_vecs, edge_index, W_src, W_dst, W_rating, a_w):
    del W_rating
    src = edge_index[0].astype(jnp.int32).reshape(NC, NS, NCH, CH)
    dst = edge_index[1].astype(jnp.int32).reshape(NC, NS, NCH, CH)
    ta, tb, tc, ssrc, sdst = _prep(source_vecs, W_src, W_dst, a_w)
    partial = _edge()(ta, tb, tc, ssrc.reshape(N), sdst.reshape(N), src, dst)
    return _post(partial)


# speedup vs baseline: 1.6371x; 1.0012x over previous
import functools

import jax
import jax.numpy as jnp
from jax import lax
from jax.experimental import pallas as pl
from jax.experimental.pallas import tpu as pltpu
from jax.experimental.pallas import tpu_sc as plsc

N = 10000
NP = N
D = 128
NT = 3
DH = 48
DC = D - 2 * DH
E = 320000
NC, NS = 2, 16
NW = NC * NS
EPT = E // NW
CH = 80
NCH = EPT // CH
RPT = 624
RB = 10
BR = N // RB

NB = 5
LOOK = 4
UNR = 8


def _prep_body(x_ref, wsrc_ref, wdst_ref, aw_ref, ta_ref, tb_ref, tc_ref,
               ssrc_ref, sdst_ref):
    x = x_ref[...]
    h = jnp.dot(x, wsrc_ref[...], preferred_element_type=jnp.float32)
    a1 = aw_ref[0:D, :]
    a2 = aw_ref[D:2 * D, :]
    ssrc_ref[...] = jnp.dot(h, a1, preferred_element_type=jnp.float32)
    w2 = jnp.dot(wdst_ref[...], a2, preferred_element_type=jnp.float32)
    sdst_ref[...] = jnp.dot(x, w2, preferred_element_type=jnp.float32)
    ta_ref[...] = h[:, 0:DH]
    tb_ref[...] = h[:, DH:2 * DH]
    col = lax.broadcasted_iota(jnp.int32, (x.shape[0], DH - DC), 1)
    pad = jnp.where(col == 0, 1.0, 0.0).astype(jnp.float32)
    tc_ref[...] = jnp.concatenate([h[:, 2 * DH:D], pad], axis=1)


_prep = pl.pallas_call(
    _prep_body,
    grid=(RB,),
    in_specs=[
        pl.BlockSpec((BR, D), lambda i: (i, 0)),
        pl.BlockSpec((D, D), lambda i: (0, 0)),
        pl.BlockSpec((D, D), lambda i: (0, 0)),
        pl.BlockSpec((3 * D, 1), lambda i: (0, 0)),
    ],
    out_specs=[
        pl.BlockSpec((BR, DH), lambda i: (i, 0)),
        pl.BlockSpec((BR, DH), lambda i: (i, 0)),
        pl.BlockSpec((BR, DH), lambda i: (i, 0)),
        pl.BlockSpec((BR, 1), lambda i: (i, 0)),
        pl.BlockSpec((BR, 1), lambda i: (i, 0)),
    ],
    out_shape=[
        jax.ShapeDtypeStruct((N, DH), jnp.float32),
        jax.ShapeDtypeStruct((N, DH), jnp.float32),
        jax.ShapeDtypeStruct((N, DH), jnp.float32),
        jax.ShapeDtypeStruct((N, 1), jnp.float32),
        jax.ShapeDtypeStruct((N, 1), jnp.float32),
    ],
)


def _edge_body(ta_hbm, tb_hbm, tc_hbm, ssrc_hbm, sdst_hbm, src_hbm, dst_hbm,
               out_hbm, ssrc_v, sdst_v, src_v, dst_v, att_v, rows_v, zero_v,
               acc_sh, sem_g, sem_s, sem_z):
    ci = lax.axis_index("c")
    si = lax.axis_index("s")

    pltpu.async_copy(ssrc_hbm, ssrc_v.at[pl.ds(0, N)], sem_z)
    pltpu.async_copy(sdst_hbm, sdst_v.at[pl.ds(0, N)], sem_z)
    pltpu.async_copy(src_hbm.at[ci, si], src_v, sem_z)
    pltpu.async_copy(dst_hbm.at[ci, si], dst_v, sem_z)

    zrow = jnp.zeros((16,), jnp.float32)
    for i in range(48):
        for q in range(DH // 16):
            zero_v[i, pl.ds(q * 16, 16)] = zrow

    pltpu.make_async_copy(ssrc_hbm, ssrc_v.at[pl.ds(0, N)], sem_z).wait()
    pltpu.make_async_copy(sdst_hbm, sdst_v.at[pl.ds(0, N)], sem_z).wait()
    pltpu.make_async_copy(src_hbm.at[ci, si], src_v, sem_z).wait()
    pltpu.make_async_copy(dst_hbm.at[ci, si], dst_v, sem_z).wait()

    base = si * RPT

    for slab, tab_hbm in ((0, ta_hbm), (1, tb_hbm), (2, tc_hbm)):
        for i in range(RPT // 48):
            pltpu.async_copy(zero_v, acc_sh.at[pl.ds(base + i * 48, 48)], sem_z)

        @pl.when(si == NS - 1)
        def _ztail():
            pltpu.async_copy(zero_v.at[pl.ds(0, NP - RPT * NS)],
                             acc_sh.at[pl.ds(RPT * NS, NP - RPT * NS)], sem_z)

        for i in range(RPT // 48):
            pltpu.make_async_copy(zero_v,
                                  acc_sh.at[pl.ds(base + i * 48, 48)],
                                  sem_z).wait()

        @pl.when(si == NS - 1)
        def _ztailw():
            pltpu.make_async_copy(zero_v.at[pl.ds(0, NP - RPT * NS)],
                                  acc_sh.at[pl.ds(RPT * NS, NP - RPT * NS)],
                                  sem_z).wait()

        plsc.subcore_barrier()

        for b in range(LOOK):
            pltpu.async_copy(tab_hbm.at[src_v.at[b]], rows_v.at[b],
                             sem_g.at[b])

        def _group(jo, carry):
            for b in range(NB):
                j = jo * NB + b
                if slab == 0:
                    for k in range(CH // 16):
                        s16 = src_v[j, pl.ds(k * 16, 16)]
                        d16 = dst_v[j, pl.ds(k * 16, 16)]
                        x = (plsc.load_gather(ssrc_v, [s16])
                             + plsc.load_gather(sdst_v, [d16]))
                        x = jnp.where(x >= 0, x, 0.2 * x)
                        att_v[j, pl.ds(k * 16, 16)] = jnp.exp(x)
                pltpu.make_async_copy(tab_hbm.at[src_v.at[j]], rows_v.at[b],
                                      sem_g.at[b]).wait()
                jj = jnp.full((16,), j, jnp.int32)

                def _mul(i, carry2):
                    e0 = i * UNR
                    for u in range(UNR):
                        ee = jnp.full((16,), e0 + u, jnp.int32)
                        a16 = plsc.load_gather(att_v, [jj, ee])
                        for q in range(DH // 16):
                            rows_v[b, e0 + u, pl.ds(q * 16, 16)] = (
                                rows_v[b, e0 + u, pl.ds(q * 16, 16)] * a16)
                    return carry2

                lax.fori_loop(0, CH // UNR, _mul, 0)
                pltpu.async_copy(rows_v.at[b], acc_sh.at[dst_v.at[j]],
                                 sem_s.at[b], add=True)
                bb = (b + LOOK) % NB

                @pl.when(j >= NB - LOOK)
                def _drain():
                    pltpu.make_async_copy(rows_v.at[bb],
                                          acc_sh.at[dst_v.at[0]],
                                          sem_s.at[bb]).wait()

                @pl.when(j + LOOK <= NCH - 1)
                def _prefetch():
                    pltpu.async_copy(tab_hbm.at[src_v.at[j + LOOK]],
                                     rows_v.at[bb], sem_g.at[bb])
            return carry

        lax.fori_loop(0, NCH // NB, _group, 0)

        for c in range(NCH - (NB - LOOK), NCH):
            pltpu.make_async_copy(rows_v.at[c % NB], acc_sh.at[dst_v.at[0]],
                                  sem_s.at[c % NB]).wait()

        plsc.subcore_barrier()

        pltpu.sync_copy(acc_sh.at[pl.ds(base, RPT)],
                        out_hbm.at[slab, ci, pl.ds(base, RPT)])

        @pl.when(si == NS - 1)
        def _tail():
            pltpu.sync_copy(acc_sh.at[pl.ds(RPT * NS, NP - RPT * NS)],
                            out_hbm.at[slab, ci, pl.ds(RPT * NS, NP - RPT * NS)])

        plsc.subcore_barrier()


@functools.cache
def _edge():
    return pl.kernel(
        _edge_body,
        out_type=jax.ShapeDtypeStruct((NT, NC, NP, DH), jnp.float32),
        mesh=plsc.VectorSubcoreMesh(core_axis_name="c", subcore_axis_name="s",
                                    num_cores=NC, num_subcores=NS),
        compiler_params=pltpu.CompilerParams(needs_layout_passes=False,
                                             use_tc_tiling_on_sc=False),
        scratch_types=[
            pltpu.VMEM((NP,), jnp.float32),
            pltpu.VMEM((NP,), jnp.float32),
            pltpu.VMEM((NCH, CH), jnp.int32),
            pltpu.VMEM((NCH, CH), jnp.int32),
            pltpu.VMEM((NCH, CH), jnp.float32),
            pltpu.VMEM((NB, CH, DH), jnp.float32),
            pltpu.VMEM((48, DH), jnp.float32),
            pltpu.VMEM_SHARED((NP, DH), jnp.float32),
            pltpu.SemaphoreType.DMA((NB,)),
            pltpu.SemaphoreType.DMA((NB,)),
            pltpu.SemaphoreType.DMA,
        ],
    )


def _post_body(p_ref, out_ref):
    a = p_ref[0, 0] + p_ref[0, 1]
    b = p_ref[1, 0] + p_ref[1, 1]
    c = p_ref[2, 0] + p_ref[2, 1]
    h = jnp.concatenate([a, b, c[:, 0:DC]], axis=1)
    sw = c[:, DC:DC + 1]
    r = h / (sw + 1e-8)
    out_ref[...] = jnp.where(r > 0, r, jnp.exp(r) - 1.0)


_post = pl.pallas_call(
    _post_body,
    grid=(RB,),
    in_specs=[pl.BlockSpec((NT, NC, BR, DH), lambda i: (0, 0, i, 0))],
    out_specs=pl.BlockSpec((BR, D), lambda i: (i, 0)),
    out_shape=jax.ShapeDtypeStruct((N, D), jnp.float32),
)


def kernel(source_vecs, edge_index, W_src, W_dst, W_rating, a_w):
    del W_rating
    src = edge_index[0].astype(jnp.int32).reshape(NC, NS, NCH, CH)
    dst = edge_index[1].astype(jnp.int32).reshape(NC, NS, NCH, CH)
    ta, tb, tc, ssrc, sdst = _prep(source_vecs, W_src, W_dst, a_w)
    partial = _edge()(ta, tb, tc, ssrc.reshape(N), sdst.reshape(N), src, dst)
    return _post(partial)


# speedup vs baseline: 1.8977x; 1.1592x over previous
import functools

import jax
import jax.numpy as jnp
from jax import lax
from jax.experimental import pallas as pl
from jax.experimental.pallas import tpu as pltpu
from jax.experimental.pallas import tpu_sc as plsc

N = 10000
D = 128
NT = 2
DH = 64
E = 320000
NC, NS = 2, 16
NW = NC * NS
EPT = E // NW
CH = 80
NCH = EPT // CH
RPT = 624
RB = 10
BR = N // RB

NB = 5
LOOK = 3
UNR = 8


def _prep_body(x_ref, wsrc_ref, wdst_ref, aw_ref, ta_ref, tb_ref,
               ssrc_ref, sdst_ref):
    x = x_ref[...]
    h = jnp.dot(x, wsrc_ref[...], preferred_element_type=jnp.float32)
    a1 = aw_ref[0:D, :]
    a2 = aw_ref[D:2 * D, :]
    ssrc_ref[...] = jnp.dot(h, a1, preferred_element_type=jnp.float32)
    w2 = jnp.dot(wdst_ref[...], a2, preferred_element_type=jnp.float32)
    sdst_ref[...] = jnp.dot(x, w2, preferred_element_type=jnp.float32)
    ta_ref[...] = h[:, 0:DH]
    tb_ref[...] = h[:, DH:D]


_prep = pl.pallas_call(
    _prep_body,
    grid=(RB,),
    in_specs=[
        pl.BlockSpec((BR, D), lambda i: (i, 0)),
        pl.BlockSpec((D, D), lambda i: (0, 0)),
        pl.BlockSpec((D, D), lambda i: (0, 0)),
        pl.BlockSpec((3 * D, 1), lambda i: (0, 0)),
    ],
    out_specs=[
        pl.BlockSpec((BR, DH), lambda i: (i, 0)),
        pl.BlockSpec((BR, DH), lambda i: (i, 0)),
        pl.BlockSpec((BR, 1), lambda i: (i, 0)),
        pl.BlockSpec((BR, 1), lambda i: (i, 0)),
    ],
    out_shape=[
        jax.ShapeDtypeStruct((N, DH), jnp.float32),
        jax.ShapeDtypeStruct((N, DH), jnp.float32),
        jax.ShapeDtypeStruct((N, 1), jnp.float32),
        jax.ShapeDtypeStruct((N, 1), jnp.float32),
    ],
)


def _edge_body(ta_hbm, tb_hbm, ssrc_hbm, sdst_hbm, src_hbm, dst_hbm,
               out_hbm, sw_hbm, ssrc_v, sdst_v, src_v, dst_v, att_v, sumw_v,
               rows_v, zero_v, acc_sh, sem_g, sem_s, sem_z):
    ci = lax.axis_index("c")
    si = lax.axis_index("s")

    pltpu.async_copy(ssrc_hbm, ssrc_v, sem_z)
    pltpu.async_copy(sdst_hbm, sdst_v, sem_z)
    pltpu.async_copy(src_hbm.at[ci, si], src_v, sem_z)
    pltpu.async_copy(dst_hbm.at[ci, si], dst_v, sem_z)

    zrow = jnp.zeros((16,), jnp.float32)
    for i in range(48):
        for q in range(DH // 16):
            zero_v[i, pl.ds(q * 16, 16)] = zrow

    def _zsw(i, carry):
        sumw_v[pl.ds(i * 16, 16)] = zrow
        return carry

    lax.fori_loop(0, N // 16, _zsw, 0)

    pltpu.make_async_copy(ssrc_hbm, ssrc_v, sem_z).wait()
    pltpu.make_async_copy(sdst_hbm, sdst_v, sem_z).wait()
    pltpu.make_async_copy(src_hbm.at[ci, si], src_v, sem_z).wait()
    pltpu.make_async_copy(dst_hbm.at[ci, si], dst_v, sem_z).wait()

    def _att_chunk(j, carry):
        for k in range(CH // 16):
            s16 = src_v[j, pl.ds(k * 16, 16)]
            d16 = dst_v[j, pl.ds(k * 16, 16)]
            x = plsc.load_gather(ssrc_v, [s16]) + plsc.load_gather(sdst_v, [d16])
            x = jnp.where(x >= 0, x, 0.2 * x)
            a = jnp.exp(x)
            att_v[j, pl.ds(k * 16, 16)] = a
            plsc.addupdate_scatter(sumw_v, [d16], a)
        return carry

    lax.fori_loop(0, NCH, _att_chunk, 0)

    base = si * RPT

    for slab, tab_hbm in ((0, ta_hbm), (1, tb_hbm)):
        for i in range(RPT // 48):
            pltpu.async_copy(zero_v, acc_sh.at[pl.ds(base + i * 48, 48)], sem_z)

        @pl.when(si == NS - 1)
        def _ztail():
            pltpu.async_copy(zero_v.at[pl.ds(0, N - RPT * NS)],
                             acc_sh.at[pl.ds(RPT * NS, N - RPT * NS)], sem_z)

        for i in range(RPT // 48):
            pltpu.make_async_copy(zero_v,
                                  acc_sh.at[pl.ds(base + i * 48, 48)],
                                  sem_z).wait()

        @pl.when(si == NS - 1)
        def _ztailw():
            pltpu.make_async_copy(zero_v.at[pl.ds(0, N - RPT * NS)],
                                  acc_sh.at[pl.ds(RPT * NS, N - RPT * NS)],
                                  sem_z).wait()

        plsc.subcore_barrier()

        for b in range(LOOK):
            pltpu.async_copy(tab_hbm.at[src_v.at[b]], rows_v.at[b],
                             sem_g.at[b])

        def _group(jo, carry):
            for b in range(NB):
                j = jo * NB + b
                pltpu.make_async_copy(tab_hbm.at[src_v.at[j]], rows_v.at[b],
                                      sem_g.at[b]).wait()
                jj = jnp.full((16,), j, jnp.int32)

                def _mul(i, carry2):
                    e0 = i * UNR
                    for u in range(UNR):
                        ee = jnp.full((16,), e0 + u, jnp.int32)
                        a16 = plsc.load_gather(att_v, [jj, ee])
                        for q in range(DH // 16):
                            rows_v[b, e0 + u, pl.ds(q * 16, 16)] = (
                                rows_v[b, e0 + u, pl.ds(q * 16, 16)] * a16)
                    return carry2

                lax.fori_loop(0, CH // UNR, _mul, 0)
                pltpu.async_copy(rows_v.at[b], acc_sh.at[dst_v.at[j]],
                                 sem_s.at[b], add=True)
                bb = (b + LOOK) % NB

                @pl.when(j >= NB - LOOK)
                def _drain():
                    pltpu.make_async_copy(rows_v.at[bb],
                                          acc_sh.at[dst_v.at[0]],
                                          sem_s.at[bb]).wait()

                @pl.when(j + LOOK <= NCH - 1)
                def _prefetch():
                    pltpu.async_copy(tab_hbm.at[src_v.at[j + LOOK]],
                                     rows_v.at[bb], sem_g.at[bb])
            return carry

        lax.fori_loop(0, NCH // NB, _group, 0)

        for c in range(NCH - (NB - LOOK), NCH):
            pltpu.make_async_copy(rows_v.at[c % NB], acc_sh.at[dst_v.at[0]],
                                  sem_s.at[c % NB]).wait()

        plsc.subcore_barrier()

        pltpu.sync_copy(acc_sh.at[pl.ds(base, RPT)],
                        out_hbm.at[slab, ci, pl.ds(base, RPT)])

        @pl.when(si == NS - 1)
        def _tail():
            pltpu.sync_copy(acc_sh.at[pl.ds(RPT * NS, N - RPT * NS)],
                            out_hbm.at[slab, ci, pl.ds(RPT * NS, N - RPT * NS)])

        plsc.subcore_barrier()

    pltpu.sync_copy(sumw_v, sw_hbm.at[ci, si])


@functools.cache
def _edge():
    return pl.kernel(
        _edge_body,
        out_type=(jax.ShapeDtypeStruct((NT, NC, N, DH), jnp.float32),
                  jax.ShapeDtypeStruct((NC, NS, N), jnp.float32)),
        mesh=plsc.VectorSubcoreMesh(core_axis_name="c", subcore_axis_name="s",
                                    num_cores=NC, num_subcores=NS),
        compiler_params=pltpu.CompilerParams(needs_layout_passes=False,
                                             use_tc_tiling_on_sc=False),
        scratch_types=[
            pltpu.VMEM((N,), jnp.float32),
            pltpu.VMEM((N,), jnp.float32),
            pltpu.VMEM((NCH, CH), jnp.int32),
            pltpu.VMEM((NCH, CH), jnp.int32),
            pltpu.VMEM((NCH, CH), jnp.float32),
            pltpu.VMEM((N,), jnp.float32),
            pltpu.VMEM((NB, CH, DH), jnp.float32),
            pltpu.VMEM((48, DH), jnp.float32),
            pltpu.VMEM_SHARED((N, DH), jnp.float32),
            pltpu.SemaphoreType.DMA((NB,)),
            pltpu.SemaphoreType.DMA((NB,)),
            pltpu.SemaphoreType.DMA,
        ],
    )


def _post_body(p_ref, sw_ref, out_ref):
    a = p_ref[0, 0] + p_ref[0, 1]
    b = p_ref[1, 0] + p_ref[1, 1]
    h = jnp.concatenate([a, b], axis=1)
    sw = jnp.sum(sw_ref[...], axis=(1, 2))[:, None]
    r = h / (sw + 1e-8)
    out_ref[...] = jnp.where(r > 0, r, jnp.exp(r) - 1.0)


_post = pl.pallas_call(
    _post_body,
    grid=(RB,),
    in_specs=[pl.BlockSpec((NT, NC, BR, DH), lambda i: (0, 0, i, 0)),
              pl.BlockSpec((BR, NC, NS), lambda i: (i, 0, 0))],
    out_specs=pl.BlockSpec((BR, D), lambda i: (i, 0)),
    out_shape=jax.ShapeDtypeStruct((N, D), jnp.float32),
)


def kernel(source_vecs, edge_index, W_src, W_dst, W_rating, a_w):
    del W_rating
    src = edge_index[0].astype(jnp.int32).reshape(NC, NS, NCH, CH)
    dst = edge_index[1].astype(jnp.int32).reshape(NC, NS, NCH, CH)
    ta, tb, ssrc, sdst = _prep(source_vecs, W_src, W_dst, a_w)
    partial, sw = _edge()(ta, tb, ssrc.reshape(N), sdst.reshape(N), src, dst)
    return _post(partial, sw.transpose(2, 0, 1))


# speedup vs baseline: 1.9826x; 1.0447x over previous
import functools

import jax
import jax.numpy as jnp
from jax import lax
from jax.experimental import pallas as pl
from jax.experimental.pallas import tpu as pltpu
from jax.experimental.pallas import tpu_sc as plsc

N = 10000
D = 128
NT = 2
DH = 64
E = 320000
NC, NS = 2, 16
NW = NC * NS
EPT = E // NW
CH = 80
NCH = EPT // CH
RPT = 624
RB = 10
BR = N // RB

NB = 5
LOOK = 3
UNR = 8


def _prep_body(x_ref, wsrc_ref, wdst_ref, aw_ref, ta_ref, tb_ref,
               ssrc_ref, sdst_ref):
    x = x_ref[...]
    h = jnp.dot(x, wsrc_ref[...], preferred_element_type=jnp.float32)
    a1 = aw_ref[0:D, :]
    a2 = aw_ref[D:2 * D, :]
    ssrc_ref[...] = jnp.dot(h, a1, preferred_element_type=jnp.float32)
    w2 = jnp.dot(wdst_ref[...], a2, preferred_element_type=jnp.float32)
    sdst_ref[...] = jnp.dot(x, w2, preferred_element_type=jnp.float32)
    ta_ref[...] = h[:, 0:DH]
    tb_ref[...] = h[:, DH:D]


_prep = pl.pallas_call(
    _prep_body,
    grid=(RB,),
    in_specs=[
        pl.BlockSpec((BR, D), lambda i: (i, 0)),
        pl.BlockSpec((D, D), lambda i: (0, 0)),
        pl.BlockSpec((D, D), lambda i: (0, 0)),
        pl.BlockSpec((3 * D, 1), lambda i: (0, 0)),
    ],
    out_specs=[
        pl.BlockSpec((BR, DH), lambda i: (i, 0)),
        pl.BlockSpec((BR, DH), lambda i: (i, 0)),
        pl.BlockSpec((BR, 1), lambda i: (i, 0)),
        pl.BlockSpec((BR, 1), lambda i: (i, 0)),
    ],
    out_shape=[
        jax.ShapeDtypeStruct((N, DH), jnp.float32),
        jax.ShapeDtypeStruct((N, DH), jnp.float32),
        jax.ShapeDtypeStruct((N, 1), jnp.float32),
        jax.ShapeDtypeStruct((N, 1), jnp.float32),
    ],
)


def _edge_body(ta_hbm, tb_hbm, ssrc_hbm, sdst_hbm, src_hbm, dst_hbm,
               out_hbm, sw_hbm, ssrc_v, sdst_v, src_v, dst_v, att_v, sumw_v,
               rows_v, zero_v, acc_sh, sem_g, sem_s, sem_z):
    ci = lax.axis_index("c")
    si = lax.axis_index("s")

    pltpu.async_copy(ssrc_hbm, ssrc_v, sem_z)
    pltpu.async_copy(sdst_hbm, sdst_v, sem_z)
    pltpu.async_copy(src_hbm.at[ci, si], src_v, sem_z)
    pltpu.async_copy(dst_hbm.at[ci, si], dst_v, sem_z)

    zrow = jnp.zeros((16,), jnp.float32)
    for i in range(48):
        for q in range(DH // 16):
            zero_v[i, pl.ds(q * 16, 16)] = zrow

    def _zsw(i, carry):
        sumw_v[pl.ds(i * 16, 16)] = zrow
        return carry

    lax.fori_loop(0, N // 16, _zsw, 0)

    pltpu.make_async_copy(ssrc_hbm, ssrc_v, sem_z).wait()
    pltpu.make_async_copy(sdst_hbm, sdst_v, sem_z).wait()
    pltpu.make_async_copy(src_hbm.at[ci, si], src_v, sem_z).wait()
    pltpu.make_async_copy(dst_hbm.at[ci, si], dst_v, sem_z).wait()

    def _att_chunk(j, carry):
        for k in range(CH // 16):
            s16 = src_v[j, pl.ds(k * 16, 16)]
            d16 = dst_v[j, pl.ds(k * 16, 16)]
            x = plsc.load_gather(ssrc_v, [s16]) + plsc.load_gather(sdst_v, [d16])
            x = jnp.where(x >= 0, x, 0.2 * x)
            a = jnp.exp(x)
            att_v[j, pl.ds(k * 16, 16)] = a
            plsc.addupdate_scatter(sumw_v, [d16], a)
        return carry

    lax.fori_loop(0, NCH, _att_chunk, 0)

    base = si * RPT

    for slab, tab_hbm in ((0, ta_hbm), (1, tb_hbm)):
        for i in range(RPT // 48):
            pltpu.async_copy(zero_v, acc_sh.at[pl.ds(base + i * 48, 48)], sem_z)

        @pl.when(si == NS - 1)
        def _ztail():
            pltpu.async_copy(zero_v.at[pl.ds(0, N - RPT * NS)],
                             acc_sh.at[pl.ds(RPT * NS, N - RPT * NS)], sem_z)

        for i in range(RPT // 48):
            pltpu.make_async_copy(zero_v,
                                  acc_sh.at[pl.ds(base + i * 48, 48)],
                                  sem_z).wait()

        @pl.when(si == NS - 1)
        def _ztailw():
            pltpu.make_async_copy(zero_v.at[pl.ds(0, N - RPT * NS)],
                                  acc_sh.at[pl.ds(RPT * NS, N - RPT * NS)],
                                  sem_z).wait()

        plsc.subcore_barrier()

        for b in range(LOOK):
            pltpu.async_copy(tab_hbm.at[src_v.at[b]], rows_v.at[b],
                             sem_g.at[b])

        def _group(jo, carry):
            for b in range(NB):
                j = jo * NB + b
                pltpu.make_async_copy(tab_hbm.at[src_v.at[j]], rows_v.at[b],
                                      sem_g.at[b]).wait()
                jj = jnp.full((16,), j, jnp.int32)

                def _mul(i, carry2):
                    e0 = i * UNR
                    for u in range(UNR):
                        ee = jnp.full((16,), e0 + u, jnp.int32)
                        a16 = plsc.load_gather(att_v, [jj, ee])
                        for q in range(DH // 16):
                            rows_v[b, e0 + u, pl.ds(q * 16, 16)] = (
                                rows_v[b, e0 + u, pl.ds(q * 16, 16)] * a16)
                    return carry2

                lax.fori_loop(0, CH // UNR, _mul, 0)
                pltpu.async_copy(rows_v.at[b], acc_sh.at[dst_v.at[j]],
                                 sem_s.at[b], add=True)
                bb = (b + LOOK) % NB

                @pl.when(j >= NB - LOOK)
                def _drain():
                    pltpu.make_async_copy(rows_v.at[bb],
                                          acc_sh.at[dst_v.at[0]],
                                          sem_s.at[bb]).wait()

                @pl.when(j + LOOK <= NCH - 1)
                def _prefetch():
                    pltpu.async_copy(tab_hbm.at[src_v.at[j + LOOK]],
                                     rows_v.at[bb], sem_g.at[bb])
            return carry

        lax.fori_loop(0, NCH // NB, _group, 0)

        for c in range(NCH - (NB - LOOK), NCH):
            pltpu.make_async_copy(rows_v.at[c % NB], acc_sh.at[dst_v.at[0]],
                                  sem_s.at[c % NB]).wait()

        plsc.subcore_barrier()

        pltpu.sync_copy(acc_sh.at[pl.ds(base, RPT)],
                        out_hbm.at[slab, ci, pl.ds(base, RPT)])

        @pl.when(si == NS - 1)
        def _tail():
            pltpu.sync_copy(acc_sh.at[pl.ds(RPT * NS, N - RPT * NS)],
                            out_hbm.at[slab, ci, pl.ds(RPT * NS, N - RPT * NS)])

        plsc.subcore_barrier()

    pltpu.sync_copy(sumw_v, sw_hbm.at[ci, si])


@functools.cache
def _edge():
    return pl.kernel(
        _edge_body,
        out_type=(jax.ShapeDtypeStruct((NT, NC, N, DH), jnp.float32),
                  jax.ShapeDtypeStruct((NC, NS, N), jnp.float32)),
        mesh=plsc.VectorSubcoreMesh(core_axis_name="c", subcore_axis_name="s",
                                    num_cores=NC, num_subcores=NS),
        compiler_params=pltpu.CompilerParams(needs_layout_passes=False,
                                             use_tc_tiling_on_sc=False),
        scratch_types=[
            pltpu.VMEM((N,), jnp.float32),
            pltpu.VMEM((N,), jnp.float32),
            pltpu.VMEM((NCH, CH), jnp.int32),
            pltpu.VMEM((NCH, CH), jnp.int32),
            pltpu.VMEM((NCH, CH), jnp.float32),
            pltpu.VMEM((N,), jnp.float32),
            pltpu.VMEM((NB, CH, DH), jnp.float32),
            pltpu.VMEM((48, DH), jnp.float32),
            pltpu.VMEM_SHARED((N, DH), jnp.float32),
            pltpu.SemaphoreType.DMA((NB,)),
            pltpu.SemaphoreType.DMA((NB,)),
            pltpu.SemaphoreType.DMA,
        ],
    )


def _post_body(p_ref, sw_ref, out_ref):
    a = p_ref[0, 0] + p_ref[0, 1]
    b = p_ref[1, 0] + p_ref[1, 1]
    h = jnp.concatenate([a, b], axis=1)
    sw = jnp.sum(sw_ref[...], axis=1)[:, None]
    r = h / (sw + 1e-8)
    out_ref[...] = jnp.where(r > 0, r, jnp.exp(r) - 1.0)


_post = pl.pallas_call(
    _post_body,
    grid=(RB,),
    in_specs=[pl.BlockSpec((NT, NC, BR, DH), lambda i: (0, 0, i, 0)),
              pl.BlockSpec((BR, NW), lambda i: (i, 0))],
    out_specs=pl.BlockSpec((BR, D), lambda i: (i, 0)),
    out_shape=jax.ShapeDtypeStruct((N, D), jnp.float32),
)


def kernel(source_vecs, edge_index, W_src, W_dst, W_rating, a_w):
    del W_rating
    src = edge_index[0].astype(jnp.int32).reshape(NC, NS, NCH, CH)
    dst = edge_index[1].astype(jnp.int32).reshape(NC, NS, NCH, CH)
    ta, tb, ssrc, sdst = _prep(source_vecs, W_src, W_dst, a_w)
    partial, sw = _edge()(ta, tb, ssrc.reshape(N), sdst.reshape(N), src, dst)
    return _post(partial, sw.transpose(2, 0, 1).reshape(N, NW))


# speedup vs baseline: 2.1137x; 1.0662x over previous
import functools

import jax
import jax.numpy as jnp
from jax import lax
from jax.experimental import pallas as pl
from jax.experimental.pallas import tpu as pltpu
from jax.experimental.pallas import tpu_sc as plsc

N = 10000
D = 128
NT = 2
DH = 64
E = 320000
NC, NS = 2, 16
NW = NC * NS
EPT = E // NW
CH = 80
NCH = EPT // CH
RPT = 624
RB = 10
BR = N // RB

NB = 5
LOOK = 3
UNR = 8


def _prep_body(x_ref, wsrc_ref, wdst_ref, aw_ref, ta_ref, tb_ref,
               ssrc_ref, sdst_ref):
    x = x_ref[...]
    h = jnp.dot(x, wsrc_ref[...], preferred_element_type=jnp.float32)
    a1 = aw_ref[0:D, :]
    a2 = aw_ref[D:2 * D, :]
    ssrc_ref[...] = jnp.dot(h, a1, preferred_element_type=jnp.float32)
    w2 = jnp.dot(wdst_ref[...], a2, preferred_element_type=jnp.float32)
    sdst_ref[...] = jnp.dot(x, w2, preferred_element_type=jnp.float32)
    ta_ref[...] = h[:, 0:DH]
    tb_ref[...] = h[:, DH:D]


_prep = pl.pallas_call(
    _prep_body,
    grid=(RB,),
    in_specs=[
        pl.BlockSpec((BR, D), lambda i: (i, 0)),
        pl.BlockSpec((D, D), lambda i: (0, 0)),
        pl.BlockSpec((D, D), lambda i: (0, 0)),
        pl.BlockSpec((3 * D, 1), lambda i: (0, 0)),
    ],
    out_specs=[
        pl.BlockSpec((BR, DH), lambda i: (i, 0)),
        pl.BlockSpec((BR, DH), lambda i: (i, 0)),
        pl.BlockSpec((BR, 1), lambda i: (i, 0)),
        pl.BlockSpec((BR, 1), lambda i: (i, 0)),
    ],
    out_shape=[
        jax.ShapeDtypeStruct((N, DH), jnp.float32),
        jax.ShapeDtypeStruct((N, DH), jnp.float32),
        jax.ShapeDtypeStruct((N, 1), jnp.float32),
        jax.ShapeDtypeStruct((N, 1), jnp.float32),
    ],
)


def _edge_body(ta_hbm, tb_hbm, ssrc_hbm, sdst_hbm, src_hbm, dst_hbm,
               out_hbm, sw_hbm, ssrc_v, sdst_v, src_v, dst_v, att_v, sumw_v,
               rows_v, zero_v, acc_sh, sem_g, sem_s, sem_z):
    ci = lax.axis_index("c")
    si = lax.axis_index("s")

    pltpu.async_copy(ssrc_hbm, ssrc_v, sem_z)
    pltpu.async_copy(sdst_hbm, sdst_v, sem_z)
    pltpu.async_copy(src_hbm.at[ci, si], src_v, sem_z)
    pltpu.async_copy(dst_hbm.at[ci, si], dst_v, sem_z)

    zrow = jnp.zeros((16,), jnp.float32)
    for i in range(48):
        for q in range(DH // 16):
            zero_v[i, pl.ds(q * 16, 16)] = zrow

    def _zsw(i, carry):
        sumw_v[pl.ds(i * 16, 16)] = zrow
        return carry

    lax.fori_loop(0, N // 16, _zsw, 0)

    pltpu.make_async_copy(ssrc_hbm, ssrc_v, sem_z).wait()
    pltpu.make_async_copy(sdst_hbm, sdst_v, sem_z).wait()
    pltpu.make_async_copy(src_hbm.at[ci, si], src_v, sem_z).wait()
    pltpu.make_async_copy(dst_hbm.at[ci, si], dst_v, sem_z).wait()

    def _att_chunk(j, carry):
        for k in range(CH // 16):
            s16 = src_v[j, pl.ds(k * 16, 16)]
            d16 = dst_v[j, pl.ds(k * 16, 16)]
            x = plsc.load_gather(ssrc_v, [s16]) + plsc.load_gather(sdst_v, [d16])
            x = jnp.where(x >= 0, x, 0.2 * x)
            a = jnp.exp(x)
            att_v[j, pl.ds(k * 16, 16)] = a
            plsc.addupdate_scatter(sumw_v, [d16], a)
        return carry

    lax.fori_loop(0, NCH, _att_chunk, 0)

    base = si * RPT

    for slab, tab_hbm in ((0, ta_hbm), (1, tb_hbm)):
        for i in range(RPT // 48):
            pltpu.async_copy(zero_v, acc_sh.at[pl.ds(base + i * 48, 48)], sem_z)

        @pl.when(si == NS - 1)
        def _ztail():
            pltpu.async_copy(zero_v.at[pl.ds(0, N - RPT * NS)],
                             acc_sh.at[pl.ds(RPT * NS, N - RPT * NS)], sem_z)

        for i in range(RPT // 48):
            pltpu.make_async_copy(zero_v,
                                  acc_sh.at[pl.ds(base + i * 48, 48)],
                                  sem_z).wait()

        @pl.when(si == NS - 1)
        def _ztailw():
            pltpu.make_async_copy(zero_v.at[pl.ds(0, N - RPT * NS)],
                                  acc_sh.at[pl.ds(RPT * NS, N - RPT * NS)],
                                  sem_z).wait()

        plsc.subcore_barrier()

        for b in range(LOOK):
            pltpu.async_copy(tab_hbm.at[src_v.at[b]], rows_v.at[b],
                             sem_g.at[b])

        def _group(jo, carry):
            for b in range(NB):
                j = jo * NB + b
                pltpu.make_async_copy(tab_hbm.at[src_v.at[j]], rows_v.at[b],
                                      sem_g.at[b]).wait()
                jj = jnp.full((16,), j, jnp.int32)

                def _mul(i, carry2):
                    e0 = i * UNR
                    for u in range(UNR):
                        ee = jnp.full((16,), e0 + u, jnp.int32)
                        a16 = plsc.load_gather(att_v, [jj, ee])
                        for q in range(DH // 16):
                            rows_v[b, e0 + u, pl.ds(q * 16, 16)] = (
                                rows_v[b, e0 + u, pl.ds(q * 16, 16)] * a16)
                    return carry2

                lax.fori_loop(0, CH // UNR, _mul, 0)
                pltpu.async_copy(rows_v.at[b], acc_sh.at[dst_v.at[j]],
                                 sem_s.at[b], add=True)
                bb = (b + LOOK) % NB

                @pl.when(j >= NB - LOOK)
                def _drain():
                    pltpu.make_async_copy(rows_v.at[bb],
                                          acc_sh.at[dst_v.at[0]],
                                          sem_s.at[bb]).wait()

                @pl.when(j + LOOK <= NCH - 1)
                def _prefetch():
                    pltpu.async_copy(tab_hbm.at[src_v.at[j + LOOK]],
                                     rows_v.at[bb], sem_g.at[bb])
            return carry

        lax.fori_loop(0, NCH // NB, _group, 0)

        for c in range(NCH - (NB - LOOK), NCH):
            pltpu.make_async_copy(rows_v.at[c % NB], acc_sh.at[dst_v.at[0]],
                                  sem_s.at[c % NB]).wait()

        plsc.subcore_barrier()

        pltpu.sync_copy(acc_sh.at[pl.ds(base, RPT)],
                        out_hbm.at[ci, pl.ds(base, RPT),
                                   pl.ds(slab * DH, DH)])

        @pl.when(si == NS - 1)
        def _tail():
            pltpu.sync_copy(acc_sh.at[pl.ds(RPT * NS, N - RPT * NS)],
                            out_hbm.at[ci, pl.ds(RPT * NS, N - RPT * NS),
                                       pl.ds(slab * DH, DH)])

        plsc.subcore_barrier()

    pltpu.sync_copy(sumw_v, sw_hbm.at[ci, si])


@functools.cache
def _edge():
    return pl.kernel(
        _edge_body,
        out_type=(jax.ShapeDtypeStruct((NC, N, D), jnp.float32),
                  jax.ShapeDtypeStruct((NC, NS, N), jnp.float32)),
        mesh=plsc.VectorSubcoreMesh(core_axis_name="c", subcore_axis_name="s",
                                    num_cores=NC, num_subcores=NS),
        compiler_params=pltpu.CompilerParams(needs_layout_passes=False,
                                             use_tc_tiling_on_sc=False),
        scratch_types=[
            pltpu.VMEM((N,), jnp.float32),
            pltpu.VMEM((N,), jnp.float32),
            pltpu.VMEM((NCH, CH), jnp.int32),
            pltpu.VMEM((NCH, CH), jnp.int32),
            pltpu.VMEM((NCH, CH), jnp.float32),
            pltpu.VMEM((N,), jnp.float32),
            pltpu.VMEM((NB, CH, DH), jnp.float32),
            pltpu.VMEM((48, DH), jnp.float32),
            pltpu.VMEM_SHARED((N, DH), jnp.float32),
            pltpu.SemaphoreType.DMA((NB,)),
            pltpu.SemaphoreType.DMA((NB,)),
            pltpu.SemaphoreType.DMA,
        ],
    )


def _post_body(p_ref, sw_ref, out_ref):
    h = p_ref[0] + p_ref[1]
    sw = jnp.sum(sw_ref[...], axis=1)[:, None]
    r = h / (sw + 1e-8)
    out_ref[...] = jnp.where(r > 0, r, jnp.exp(r) - 1.0)


_post = pl.pallas_call(
    _post_body,
    grid=(RB,),
    in_specs=[pl.BlockSpec((NC, BR, D), lambda i: (0, i, 0)),
              pl.BlockSpec((BR, NW), lambda i: (i, 0))],
    out_specs=pl.BlockSpec((BR, D), lambda i: (i, 0)),
    out_shape=jax.ShapeDtypeStruct((N, D), jnp.float32),
)


def kernel(source_vecs, edge_index, W_src, W_dst, W_rating, a_w):
    del W_rating
    src = edge_index[0].astype(jnp.int32).reshape(NC, NS, NCH, CH)
    dst = edge_index[1].astype(jnp.int32).reshape(NC, NS, NCH, CH)
    ta, tb, ssrc, sdst = _prep(source_vecs, W_src, W_dst, a_w)
    partial, sw = _edge()(ta, tb, ssrc.reshape(N), sdst.reshape(N), src, dst)
    return _post(partial, sw.transpose(2, 0, 1).reshape(N, NW))
